# Initial kernel scaffold; baseline (speedup 1.0000x reference)
import jax
import jax.numpy as jnp
from jax.experimental import pallas as pl


def kernel(x, vals, cheby_coeffs, t_mid, t_half, rows, cols):
    raise NotImplementedError("write your pallas kernel here")


# speedup vs baseline: 17.6324x; 17.6324x over previous
import functools

import jax
import jax.numpy as jnp
from jax import lax
from jax.experimental import pallas as pl
from jax.experimental.pallas import tpu as pltpu
from jax.experimental.pallas import tpu_sc as plsc

_NU = 4096
_NI = 4096
_DEG = 16

_NTILE = 16
_DROWS = 84
_C = _DROWS * 128
_NNZ_PAD = _NTILE * _C
_WIN = 256
_WWORDS = _WIN * _NI
_NPASS = (_NU // 2) // _WIN
_TSLICE = _WWORDS // _NTILE
_ZCH = 8192


def _densify(rows, cols, vals, zeros):
    mesh = plsc.VectorSubcoreMesh(core_axis_name="c", subcore_axis_name="s")

    @functools.partial(
        pl.kernel,
        out_type=jax.ShapeDtypeStruct((_NU * _NI,), jnp.float32),
        mesh=mesh,
        scratch_types=[
            pltpu.VMEM((_C,), jnp.int32),
            pltpu.VMEM((_C,), jnp.int32),
            pltpu.VMEM((_C,), jnp.float32),
            pltpu.VMEM((_DROWS, 128), jnp.int32),
            pltpu.VMEM((_DROWS, 128), jnp.float32),
            pltpu.VMEM((_ZCH,), jnp.float32),
            pltpu.VMEM_SHARED((_WWORDS,), jnp.float32),
            pltpu.SemaphoreType.DMA,
            pltpu.SemaphoreType.DMA,
            pltpu.SemaphoreType.DMA,
            pltpu.SemaphoreType.DMA,
        ],
    )
    def k(rows_hbm, cols_hbm, vals_hbm, zeros_hbm, out_hbm,
          rbuf, cbuf, vbuf, ibuf, ovbuf, zbuf, spw, sm0, sm1, sm2, sm3):
        c = lax.axis_index("c")
        s = lax.axis_index("s")
        base = s * _C
        pltpu.sync_copy(rows_hbm.at[pl.ds(base, _C)], rbuf)
        pltpu.sync_copy(cols_hbm.at[pl.ds(base, _C)], cbuf)
        pltpu.sync_copy(vals_hbm.at[pl.ds(base, _C)], vbuf)
        pltpu.sync_copy(zeros_hbm, zbuf)

        def gf_body(i, carry):
            sl = pl.ds(i * 16, 16)
            rbuf[sl] = rbuf[sl] * _NI + cbuf[sl]
            return carry

        lax.fori_loop(0, _C // 16, gf_body, 0)

        def pass_body(p, carry):
            lo = c * (_NU // 2) + p * _WIN
            off = lo * _NI

            for z in range(_TSLICE // _ZCH):
                pltpu.sync_copy(
                    zbuf, spw.at[pl.ds(s * _TSLICE + z * _ZCH, _ZCH)])
            plsc.subcore_barrier()

            def build(i, carry2):
                sl = pl.ds(i * 16, 16)
                d = rbuf[sl] - off
                inw = (d >= 0) & (d < _WWORDS)
                jj = i // 8
                ll = (i % 8) * 16
                ibuf[jj, pl.ds(ll, 16)] = jnp.where(inw, d, 0)
                ovbuf[jj, pl.ds(ll, 16)] = jnp.where(inw, vbuf[sl], 0.0)
                return carry2

            lax.fori_loop(0, _C // 16, build, 0)

            def fire(q, carry2):
                j = q * 4
                d0 = pltpu.async_copy(
                    ovbuf.at[j], spw.at[ibuf.at[j]], sm0, add=True)
                d1 = pltpu.async_copy(
                    ovbuf.at[j + 1], spw.at[ibuf.at[j + 1]], sm1, add=True)
                d2 = pltpu.async_copy(
                    ovbuf.at[j + 2], spw.at[ibuf.at[j + 2]], sm2, add=True)
                d3 = pltpu.async_copy(
                    ovbuf.at[j + 3], spw.at[ibuf.at[j + 3]], sm3, add=True)
                d0.wait()
                d1.wait()
                d2.wait()
                d3.wait()
                return carry2

            lax.fori_loop(0, _DROWS // 4, fire, 0)
            plsc.subcore_barrier()

            pltpu.sync_copy(
                spw.at[pl.ds(s * _TSLICE, _TSLICE)],
                out_hbm.at[pl.ds(off + s * _TSLICE, _TSLICE)])
            plsc.subcore_barrier()
            return carry

        lax.fori_loop(0, _NPASS, pass_body, 0)

    return k(rows, cols, vals, zeros)


_XBLK = 1024
_NBLK = _NU // _XBLK


def _cheby_body(x_blk_ref, s0_ref, co_ref, mid_ref, half_ref, out_ref,
                sp, sc, gacc, acc):
    k = pl.program_id(0)
    j = pl.program_id(1)

    @pl.when((k == 0) & (j == 0))
    def _init():
        s0 = s0_ref[...]
        sp[...] = s0
        sc[...] = s0
        acc[...] = co_ref[0] * s0

    @pl.when(j == 0)
    def _zero():
        gacc[...] = jnp.zeros_like(gacc)

    xb = x_blk_ref[...]
    u = lax.dot_general(sc[...], xb, (((1,), (1,)), ((), ())),
                        preferred_element_type=jnp.float32)
    gacc[...] += lax.dot_general(u, xb, (((1,), (0,)), ((), ())),
                                 preferred_element_type=jnp.float32)

    @pl.when(j == _NBLK - 1)
    def _finish():
        mid = mid_ref[0]
        inv_half = 1.0 / half_ref[0]
        t = (gacc[...] - mid * sc[...]) * inv_half
        sn = jnp.where(k == 0, t, 2.0 * t - sp[...])
        acc[...] += co_ref[k + 1] * sn
        sp[...] = sc[...]
        sc[...] = sn

        @pl.when(k == _DEG - 1)
        def _emit():
            out_ref[...] = acc[...]


def _cheby(xd, xb, coeffs, mid, half):
    b = xb.shape[0]
    return pl.pallas_call(
        _cheby_body,
        grid=(_DEG, _NBLK),
        out_shape=jax.ShapeDtypeStruct((b, _NI), jnp.float32),
        in_specs=[
            pl.BlockSpec((_XBLK, _NI), lambda k, j: (j, 0)),
            pl.BlockSpec((b, _NI), lambda k, j: (0, 0)),
            pl.BlockSpec(memory_space=pltpu.SMEM),
            pl.BlockSpec(memory_space=pltpu.SMEM),
            pl.BlockSpec(memory_space=pltpu.SMEM),
        ],
        out_specs=pl.BlockSpec((b, _NI), lambda k, j: (0, 0)),
        scratch_shapes=[
            pltpu.VMEM((b, _NI), jnp.float32),
            pltpu.VMEM((b, _NI), jnp.float32),
            pltpu.VMEM((b, _NI), jnp.float32),
            pltpu.VMEM((b, _NI), jnp.float32),
        ],
    )(xd, xb, coeffs, mid, half)


def kernel(x, vals, cheby_coeffs, t_mid, t_half, rows, cols):
    nnz = rows.shape[0]
    pad = _NNZ_PAD - nnz
    rows_p = jnp.concatenate([rows, jnp.zeros((pad,), jnp.int32)])
    cols_p = jnp.concatenate([cols, jnp.zeros((pad,), jnp.int32)])
    vals_p = jnp.concatenate([vals, jnp.zeros((pad,), jnp.float32)])
    zeros = jnp.zeros((_ZCH,), jnp.float32)

    xflat = _densify(rows_p, cols_p, vals_p, zeros)
    xd = xflat.reshape(_NU, _NI)

    mid = jnp.reshape(t_mid.astype(jnp.float32), (1,))
    half = jnp.reshape(t_half.astype(jnp.float32), (1,))
    return _cheby(xd, x, cheby_coeffs.astype(jnp.float32), mid, half)


# speedup vs baseline: 59.4735x; 3.3730x over previous
import functools

import jax
import jax.numpy as jnp
from jax import lax
from jax.experimental import pallas as pl
from jax.experimental.pallas import tpu as pltpu
from jax.experimental.pallas import tpu_sc as plsc

_NU = 4096
_NI = 4096
_DEG = 16

_NTILE = 16
_DROWS = 84
_C = _DROWS * 128
_NNZ_PAD = _NTILE * _C
_WIN = 256
_WWORDS = _WIN * _NI
_NPASS = (_NU // 2) // _WIN
_TSLICE = _WWORDS // _NTILE
_ZCH = 8192


def _densify(rows, cols, vals, zeros):
    mesh = plsc.VectorSubcoreMesh(core_axis_name="c", subcore_axis_name="s")

    @functools.partial(
        pl.kernel,
        out_type=jax.ShapeDtypeStruct((_NU * _NI,), jnp.float32),
        mesh=mesh,
        scratch_types=[
            pltpu.VMEM((_C,), jnp.int32),
            pltpu.VMEM((_C,), jnp.int32),
            pltpu.VMEM((_C,), jnp.float32),
            pltpu.VMEM((_DROWS, 128), jnp.int32),
            pltpu.VMEM((_DROWS, 128), jnp.float32),
            pltpu.VMEM((_ZCH,), jnp.float32),
            pltpu.VMEM_SHARED((_WWORDS,), jnp.float32),
            pltpu.SemaphoreType.DMA,
            pltpu.SemaphoreType.DMA,
            pltpu.SemaphoreType.DMA,
            pltpu.SemaphoreType.DMA,
        ],
    )
    def k(rows_hbm, cols_hbm, vals_hbm, zeros_hbm, out_hbm,
          rbuf, cbuf, vbuf, ibuf, ovbuf, zbuf, spw, sm0, sm1, sm2, sm3):
        c = lax.axis_index("c")
        s = lax.axis_index("s")
        base = s * _C
        pltpu.sync_copy(rows_hbm.at[pl.ds(base, _C)], rbuf)
        pltpu.sync_copy(cols_hbm.at[pl.ds(base, _C)], cbuf)
        pltpu.sync_copy(vals_hbm.at[pl.ds(base, _C)], vbuf)
        pltpu.sync_copy(zeros_hbm, zbuf)

        def gf_body(i, carry):
            sl = pl.ds(i * 16, 16)
            rbuf[sl] = rbuf[sl] * _NI + cbuf[sl]
            return carry

        lax.fori_loop(0, _C // 16, gf_body, 0)

        def pass_body(p, carry):
            lo = c * (_NU // 2) + p * _WIN
            off = lo * _NI

            for z in range(_TSLICE // _ZCH):
                pltpu.sync_copy(
                    zbuf, spw.at[pl.ds(s * _TSLICE + z * _ZCH, _ZCH)])
            plsc.subcore_barrier()

            def build(i, carry2):
                sl = pl.ds(i * 16, 16)
                d = rbuf[sl] - off
                inw = (d >= 0) & (d < _WWORDS)
                jj = i // 8
                ll = (i % 8) * 16
                ibuf[jj, pl.ds(ll, 16)] = d & (_WWORDS - 1)
                ovbuf[jj, pl.ds(ll, 16)] = jnp.where(inw, vbuf[sl], 0.0)
                return carry2

            lax.fori_loop(0, _C // 16, build, 0)

            def fire(q, carry2):
                j = q * 4
                d0 = pltpu.async_copy(
                    ovbuf.at[j], spw.at[ibuf.at[j]], sm0, add=True)
                d1 = pltpu.async_copy(
                    ovbuf.at[j + 1], spw.at[ibuf.at[j + 1]], sm1, add=True)
                d2 = pltpu.async_copy(
                    ovbuf.at[j + 2], spw.at[ibuf.at[j + 2]], sm2, add=True)
                d3 = pltpu.async_copy(
                    ovbuf.at[j + 3], spw.at[ibuf.at[j + 3]], sm3, add=True)
                d0.wait()
                d1.wait()
                d2.wait()
                d3.wait()
                return carry2

            lax.fori_loop(0, _DROWS // 4, fire, 0)
            plsc.subcore_barrier()

            pltpu.sync_copy(
                spw.at[pl.ds(s * _TSLICE, _TSLICE)],
                out_hbm.at[pl.ds(off + s * _TSLICE, _TSLICE)])
            plsc.subcore_barrier()
            return carry

        lax.fori_loop(0, _NPASS, pass_body, 0)

    return k(rows, cols, vals, zeros)


_XBLK = 1024
_NBLK = _NU // _XBLK


def _cheby_body(x_blk_ref, s0_ref, co_ref, mid_ref, half_ref, out_ref,
                sp, sc, gacc, acc):
    k = pl.program_id(0)
    j = pl.program_id(1)

    @pl.when((k == 0) & (j == 0))
    def _init():
        s0 = s0_ref[...]
        sp[...] = s0
        sc[...] = s0
        acc[...] = co_ref[0] * s0

    @pl.when(j == 0)
    def _zero():
        gacc[...] = jnp.zeros_like(gacc)

    xb = x_blk_ref[...]
    u = lax.dot_general(sc[...], xb, (((1,), (1,)), ((), ())),
                        preferred_element_type=jnp.float32)
    gacc[...] += lax.dot_general(u, xb, (((1,), (0,)), ((), ())),
                                 preferred_element_type=jnp.float32)

    @pl.when(j == _NBLK - 1)
    def _finish():
        mid = mid_ref[0]
        inv_half = 1.0 / half_ref[0]
        t = (gacc[...] - mid * sc[...]) * inv_half
        sn = jnp.where(k == 0, t, 2.0 * t - sp[...])
        acc[...] += co_ref[k + 1] * sn
        sp[...] = sc[...]
        sc[...] = sn

        @pl.when(k == _DEG - 1)
        def _emit():
            out_ref[...] = acc[...]


def _cheby(xd, xb, coeffs, mid, half):
    b = xb.shape[0]
    return pl.pallas_call(
        _cheby_body,
        grid=(_DEG, _NBLK),
        out_shape=jax.ShapeDtypeStruct((b, _NI), jnp.float32),
        in_specs=[
            pl.BlockSpec((_XBLK, _NI), lambda k, j: (j, 0)),
            pl.BlockSpec((b, _NI), lambda k, j: (0, 0)),
            pl.BlockSpec(memory_space=pltpu.SMEM),
            pl.BlockSpec(memory_space=pltpu.SMEM),
            pl.BlockSpec(memory_space=pltpu.SMEM),
        ],
        out_specs=pl.BlockSpec((b, _NI), lambda k, j: (0, 0)),
        scratch_shapes=[
            pltpu.VMEM((b, _NI), jnp.float32),
            pltpu.VMEM((b, _NI), jnp.float32),
            pltpu.VMEM((b, _NI), jnp.float32),
            pltpu.VMEM((b, _NI), jnp.float32),
        ],
    )(xd, xb, coeffs, mid, half)


def kernel(x, vals, cheby_coeffs, t_mid, t_half, rows, cols):
    nnz = rows.shape[0]
    pad = _NNZ_PAD - nnz
    rows_p = jnp.concatenate([rows, jnp.zeros((pad,), jnp.int32)])
    cols_p = jnp.concatenate([cols, jnp.zeros((pad,), jnp.int32)])
    vals_p = jnp.concatenate([vals, jnp.zeros((pad,), jnp.float32)])
    zeros = jnp.zeros((_ZCH,), jnp.float32)

    xflat = _densify(rows_p, cols_p, vals_p, zeros)
    xd = xflat.reshape(_NU, _NI)

    mid = jnp.reshape(t_mid.astype(jnp.float32), (1,))
    half = jnp.reshape(t_half.astype(jnp.float32), (1,))
    return _cheby(xd, x, cheby_coeffs.astype(jnp.float32), mid, half)
